# baseline (speedup 1.0000x reference)
import functools

import jax
import jax.numpy as jnp
from jax import lax
from jax.experimental import pallas as pl
from jax.experimental.pallas import tpu as pltpu
from jax.experimental.pallas import tpu_sc as plsc

N = 10000
E = 320000
C = 128
BN_EPS = 1e-5

NC, NS, L = 2, 16, 16
NW = NC * NS
EPW = E // NW
G = C // L
NPAD = 10112
ROWS_PER_TILE = NPAD // NS

KA = 80
CHUNKS_A = EPW // KA
PB = 8
LEAD_P = 4
LEAD_Q = 2
KB = 40
CHUNKS_B = EPW // KB
NBUF = 2
NAB = 2
NIDX = 4

_mesh = plsc.VectorSubcoreMesh(core_axis_name="c", subcore_axis_name="s")


def _wid():
    return lax.axis_index("s") * NC + lax.axis_index("c")


def _pq_body(x_ref, w_ref, p_ref, q_ref):
    wt = w_ref[0:C, :]
    wb = w_ref[C : 2 * C, :]
    xb = x_ref[...]
    p_ref[...] = jnp.dot(xb, wt - wb, preferred_element_type=jnp.float32)
    q_ref[...] = jnp.dot(xb, wb, preferred_element_type=jnp.float32)


def _make_pq(x, w):
    blk = 1000
    return pl.pallas_call(
        _pq_body,
        grid=(N // blk,),
        in_specs=[
            pl.BlockSpec((blk, C), lambda i: (i, 0)),
            pl.BlockSpec((2 * C, C), lambda i: (0, 0)),
        ],
        out_specs=[
            pl.BlockSpec((blk, C), lambda i: (i, 0)),
            pl.BlockSpec((blk, C), lambda i: (i, 0)),
        ],
        out_shape=[
            jax.ShapeDtypeStruct((N, C), jnp.float32),
            jax.ShapeDtypeStruct((N, C), jnp.float32),
        ],
    )(x, w)


def _ssmake_body(st_ref, g_ref, b_ref, ss_ref):
    tot = jnp.sum(st_ref[...], axis=0)
    mean = tot[0:1] * (1.0 / E)
    var = tot[1:2] * (1.0 / E) - mean * mean
    scale = g_ref[...] * lax.rsqrt(var + BN_EPS)
    shift = b_ref[...] - mean * scale
    ss_ref[0:1, :] = -scale
    ss_ref[1:2, :] = -shift


def _ssmake(stats, gamma, beta):
    return pl.pallas_call(
        _ssmake_body,
        grid=(1,),
        in_specs=[
            pl.BlockSpec((NW, 2, C), lambda i: (0, 0, 0)),
            pl.BlockSpec((1, C), lambda i: (0, 0)),
            pl.BlockSpec((1, C), lambda i: (0, 0)),
        ],
        out_specs=pl.BlockSpec((2, C), lambda i: (0, 0)),
        out_shape=jax.ShapeDtypeStruct((2, C), jnp.float32),
    )(stats, gamma, beta)


@functools.partial(
    pl.kernel,
    mesh=_mesh,
    out_type=[
        jax.ShapeDtypeStruct((NW, 2, C), jnp.float32),
        jax.ShapeDtypeStruct((NW, EPW, C), jnp.float32),
    ],
    scratch_types=[
        pltpu.VMEM((CHUNKS_A, KA), jnp.int32),
        pltpu.VMEM((CHUNKS_A, KA), jnp.int32),
        pltpu.VMEM((PB, KA, C), jnp.float32),
        pltpu.VMEM((2, C), jnp.float32),
        pltpu.SemaphoreType.DMA,
        pltpu.SemaphoreType.DMA,
        pltpu.SemaphoreType.DMA,
    ],
)
def _stats_kernel(row_hbm, col_hbm, p_hbm, q_hbm, out_hbm, h_hbm,
                  idx_r, idx_c, buf, stat_v, sem_p, sem_q, sem_w):
    wid = _wid()
    pltpu.sync_copy(row_hbm.at[wid], idx_r)
    pltpu.sync_copy(col_hbm.at[wid], idx_c)

    def issue_p(c, s):
        pltpu.async_copy(p_hbm.at[idx_r.at[c]], buf.at[s], sem_p)

    def issue_q(c, s):
        pltpu.async_copy(q_hbm.at[idx_c.at[c]], buf.at[s], sem_q, add=True)

    def drain_p():
        pltpu.make_async_copy(p_hbm.at[idx_r.at[0]], buf.at[0], sem_p).wait()

    def drain_q():
        pltpu.make_async_copy(q_hbm.at[idx_c.at[0]], buf.at[0], sem_q).wait()

    def issue_w(c, s):
        base = pl.multiple_of(c * KA, KA)
        pltpu.async_copy(buf.at[s], h_hbm.at[wid, pl.ds(base, KA)], sem_w)

    def drain_w():
        pltpu.make_async_copy(
            buf.at[0], h_hbm.at[0, pl.ds(0, KA)], sem_w
        ).wait()

    def make_edge(s):
        def edge(j, accs):
            out = []
            for g in range(G):
                sl = pl.ds(g * L, L)
                h = buf[s, j, sl]
                out.append(accs[g] + h)
                out.append(accs[G + g] + h * h)
            return tuple(out[0::2] + out[1::2])

        return edge

    def body(c, k, accs):
        s_c = k % PB
        s_p = (k + LEAD_P) % PB
        s_q = (k + LEAD_Q) % PB

        @pl.when(c >= LEAD_P)
        def _():
            drain_w()

        @pl.when(c + LEAD_P < CHUNKS_A)
        def _():
            issue_p(c + LEAD_P, s_p)

        @pl.when(c + LEAD_Q < CHUNKS_A)
        def _():
            drain_p()
            issue_q(c + LEAD_Q, s_q)

        drain_q()
        accs = lax.fori_loop(0, KA, make_edge(s_c), accs)
        issue_w(c, s_c)
        return accs

    for c0 in range(LEAD_P):
        issue_p(c0, c0)
    for c0 in range(LEAD_Q):
        drain_p()
        issue_q(c0, c0)

    def outer(i, accs):
        for k in range(PB):
            accs = body(i * PB + k, k, accs)
        return accs

    zero = jnp.zeros((L,), jnp.float32)
    accs = tuple(zero for _ in range(2 * G))
    accs = lax.fori_loop(0, CHUNKS_A // PB, outer, accs)
    for t in range((CHUNKS_A // PB) * PB, CHUNKS_A):
        accs = body(t, t % PB, accs)
    for _ in range(LEAD_P):
        drain_w()

    for g in range(G):
        stat_v[0, pl.ds(g * L, L)] = accs[g]
        stat_v[1, pl.ds(g * L, L)] = accs[G + g]
    pltpu.sync_copy(stat_v, out_hbm.at[wid])


@functools.partial(
    pl.kernel,
    mesh=_mesh,
    out_type=jax.ShapeDtypeStruct((NC, NPAD, C), jnp.float32),
    scratch_types=[
        pltpu.VMEM((NIDX, KB), jnp.int32),
        pltpu.VMEM((NBUF, KB, C), jnp.float32),
        pltpu.VMEM((NAB, KB, C), jnp.float32),
        pltpu.VMEM((2, C), jnp.float32),
        pltpu.VMEM_SHARED((NPAD, C), jnp.float32),
        pltpu.SemaphoreType.DMA,
        pltpu.SemaphoreType.DMA,
        pltpu.SemaphoreType.DMA,
    ],
)
def _edge_kernel(ri_hbm, h_hbm, ss_hbm, z_hbm, out_hbm,
                 idx, buf_h, act, ss_v, accum,
                 sem_i, sem_g, sem_s):
    cid = lax.axis_index("c")
    sid = lax.axis_index("s")
    wid = sid * NC + cid

    pltpu.sync_copy(ss_hbm, ss_v)
    svals = [ss_v[0, pl.ds(g * L, L)] for g in range(G)]
    tvals = [ss_v[1, pl.ds(g * L, L)] for g in range(G)]

    pltpu.sync_copy(
        z_hbm.at[pl.ds(sid * ROWS_PER_TILE, ROWS_PER_TILE)],
        accum.at[pl.ds(sid * ROWS_PER_TILE, ROWS_PER_TILE)],
    )
    plsc.subcore_barrier()

    def issue_idx(c, ib):
        pltpu.async_copy(ri_hbm.at[wid, c], idx.at[pl.ds(ib, 1)], sem_i)

    def drain_idx():
        pltpu.make_async_copy(ri_hbm.at[0, 0], idx.at[pl.ds(0, 1)],
                              sem_i).wait()

    def issue_read(c, b):
        base = pl.multiple_of(c * KB, KB)
        pltpu.async_copy(h_hbm.at[wid, pl.ds(base, KB)], buf_h.at[b], sem_g)

    def drain_read(b):
        pltpu.make_async_copy(
            h_hbm.at[0, pl.ds(0, KB)], buf_h.at[b], sem_g
        ).wait()

    def drain_scatter(sb):
        pltpu.make_async_copy(
            act.at[sb], accum.at[idx.at[0]], sem_s
        ).wait()

    def make_edge(b, sb):
        def edge(j, _):
            for g in range(G):
                sl = pl.ds(g * L, L)
                z = buf_h[b, j, sl] * svals[g] + tvals[g]
                act[sb, j, sl] = z / (-1.0 - jnp.exp(z))
            return 0

        return edge

    for c0 in range(NIDX):
        issue_idx(c0, c0)
    issue_read(0, 0)
    issue_read(1, 1)

    def body(c, u):
        b = u % NBUF
        ib = u % NIDX
        sb = u % NAB

        drain_read(b)

        @pl.when(c >= NAB)
        def _():
            drain_scatter(sb)

            @pl.when(c + NIDX - NAB < CHUNKS_B)
            def _():
                issue_idx(c + NIDX - NAB, (c + NIDX - NAB) % NIDX)

        lax.fori_loop(0, KB, make_edge(b, sb), 0)

        @pl.when(c + NBUF < CHUNKS_B)
        def _():
            issue_read(c + NBUF, b)

        drain_idx()
        pltpu.async_copy(act.at[sb], accum.at[idx.at[ib]], sem_s, add=True)

    def outer(i, _):
        for u in range(NIDX):
            body(i * NIDX + u, u)
        return 0

    lax.fori_loop(0, CHUNKS_B // NIDX, outer, 0)
    for t in range((CHUNKS_B // NIDX) * NIDX, CHUNKS_B):
        body(t, t % NIDX)
    for sb in range(NAB):
        drain_scatter(sb)

    plsc.subcore_barrier()
    pltpu.sync_copy(
        accum.at[pl.ds(sid * ROWS_PER_TILE, ROWS_PER_TILE)],
        out_hbm.at[cid, pl.ds(sid * ROWS_PER_TILE, ROWS_PER_TILE)],
    )


def _fold_body(part_ref, out_ref):
    out_ref[...] = part_ref[0] + part_ref[1]


def _fold(partials):
    blk = 1000
    return pl.pallas_call(
        _fold_body,
        grid=(N // blk,),
        in_specs=[pl.BlockSpec((NC, blk, C), lambda i: (0, i, 0))],
        out_specs=pl.BlockSpec((blk, C), lambda i: (i, 0)),
        out_shape=jax.ShapeDtypeStruct((N, C), jnp.float32),
    )(partials)


def kernel(x_bk_c, edge_index_batched, W, gamma, beta):
    row3a = edge_index_batched[0].reshape(NW, CHUNKS_A, KA)
    col3a = edge_index_batched[1].reshape(NW, CHUNKS_A, KA)
    ri4 = edge_index_batched[0].reshape(NW, CHUNKS_B, 1, KB)
    p_tab, q_tab = _make_pq(x_bk_c, W)

    stats, h = _stats_kernel(row3a, col3a, p_tab, q_tab)
    ss = _ssmake(stats, gamma.reshape(1, C), beta.reshape(1, C))

    zeros = jnp.zeros((NPAD, C), jnp.float32)
    partials = _edge_kernel(ri4, h, ss, zeros)
    return _fold(partials)


# speedup vs baseline: 1.0978x; 1.0978x over previous
import functools

import jax
import jax.numpy as jnp
from jax import lax
from jax.experimental import pallas as pl
from jax.experimental.pallas import tpu as pltpu
from jax.experimental.pallas import tpu_sc as plsc

N = 10000
E = 320000
C = 128
BN_EPS = 1e-5

NC, NS, L = 2, 16, 16
NW = NC * NS
EPW = E // NW
G = C // L
NPAD = 10112
ROWS_PER_TILE = NPAD // NS

KA = 80
CHUNKS_A = EPW // KA
PB = 8
LEAD_P = 4
LEAD_Q = 2
KB = 40
CHUNKS_B = EPW // KB
NBUF = 4
NAB = 4
NIDX = 8

_mesh = plsc.VectorSubcoreMesh(core_axis_name="c", subcore_axis_name="s")


def _wid():
    return lax.axis_index("s") * NC + lax.axis_index("c")


def _pq_body(x_ref, w_ref, p_ref, q_ref):
    wt = w_ref[0:C, :]
    wb = w_ref[C : 2 * C, :]
    xb = x_ref[...]
    p_ref[...] = jnp.dot(xb, wt - wb, preferred_element_type=jnp.float32)
    q_ref[...] = jnp.dot(xb, wb, preferred_element_type=jnp.float32)


def _make_pq(x, w):
    blk = 1000
    return pl.pallas_call(
        _pq_body,
        grid=(N // blk,),
        in_specs=[
            pl.BlockSpec((blk, C), lambda i: (i, 0)),
            pl.BlockSpec((2 * C, C), lambda i: (0, 0)),
        ],
        out_specs=[
            pl.BlockSpec((blk, C), lambda i: (i, 0)),
            pl.BlockSpec((blk, C), lambda i: (i, 0)),
        ],
        out_shape=[
            jax.ShapeDtypeStruct((N, C), jnp.float32),
            jax.ShapeDtypeStruct((N, C), jnp.float32),
        ],
    )(x, w)


def _ssmake_body(st_ref, g_ref, b_ref, ss_ref):
    tot = jnp.sum(st_ref[...], axis=0)
    mean = tot[0:1] * (1.0 / E)
    var = tot[1:2] * (1.0 / E) - mean * mean
    scale = g_ref[...] * lax.rsqrt(var + BN_EPS)
    shift = b_ref[...] - mean * scale
    ss_ref[0:1, :] = -scale
    ss_ref[1:2, :] = -shift


def _ssmake(stats, gamma, beta):
    return pl.pallas_call(
        _ssmake_body,
        grid=(1,),
        in_specs=[
            pl.BlockSpec((NW, 2, C), lambda i: (0, 0, 0)),
            pl.BlockSpec((1, C), lambda i: (0, 0)),
            pl.BlockSpec((1, C), lambda i: (0, 0)),
        ],
        out_specs=pl.BlockSpec((2, C), lambda i: (0, 0)),
        out_shape=jax.ShapeDtypeStruct((2, C), jnp.float32),
    )(stats, gamma, beta)


@functools.partial(
    pl.kernel,
    mesh=_mesh,
    out_type=[
        jax.ShapeDtypeStruct((NW, 2, C), jnp.float32),
        jax.ShapeDtypeStruct((NW, EPW, C), jnp.float32),
    ],
    scratch_types=[
        pltpu.VMEM((CHUNKS_A, KA), jnp.int32),
        pltpu.VMEM((CHUNKS_A, KA), jnp.int32),
        pltpu.VMEM((PB, KA, C), jnp.float32),
        pltpu.VMEM((2, C), jnp.float32),
        pltpu.SemaphoreType.DMA,
        pltpu.SemaphoreType.DMA,
        pltpu.SemaphoreType.DMA,
    ],
)
def _stats_kernel(row_hbm, col_hbm, p_hbm, q_hbm, out_hbm, h_hbm,
                  idx_r, idx_c, buf, stat_v, sem_p, sem_q, sem_w):
    wid = _wid()
    pltpu.sync_copy(row_hbm.at[wid], idx_r)
    pltpu.sync_copy(col_hbm.at[wid], idx_c)

    def issue_p(c, s):
        pltpu.async_copy(p_hbm.at[idx_r.at[c]], buf.at[s], sem_p)

    def issue_q(c, s):
        pltpu.async_copy(q_hbm.at[idx_c.at[c]], buf.at[s], sem_q, add=True)

    def drain_p():
        pltpu.make_async_copy(p_hbm.at[idx_r.at[0]], buf.at[0], sem_p).wait()

    def drain_q():
        pltpu.make_async_copy(q_hbm.at[idx_c.at[0]], buf.at[0], sem_q).wait()

    def issue_w(c, s):
        base = pl.multiple_of(c * KA, KA)
        pltpu.async_copy(buf.at[s], h_hbm.at[wid, pl.ds(base, KA)], sem_w)

    def drain_w():
        pltpu.make_async_copy(
            buf.at[0], h_hbm.at[0, pl.ds(0, KA)], sem_w
        ).wait()

    def make_edge(s):
        def edge(j, accs):
            out = []
            for g in range(G):
                sl = pl.ds(g * L, L)
                h = buf[s, j, sl]
                out.append(accs[g] + h)
                out.append(accs[G + g] + h * h)
            return tuple(out[0::2] + out[1::2])

        return edge

    def body(c, k, accs):
        s_c = k % PB
        s_p = (k + LEAD_P) % PB
        s_q = (k + LEAD_Q) % PB

        @pl.when(c >= LEAD_P)
        def _():
            drain_w()

        @pl.when(c + LEAD_P < CHUNKS_A)
        def _():
            issue_p(c + LEAD_P, s_p)

        @pl.when(c + LEAD_Q < CHUNKS_A)
        def _():
            drain_p()
            issue_q(c + LEAD_Q, s_q)

        drain_q()
        accs = lax.fori_loop(0, KA, make_edge(s_c), accs)
        issue_w(c, s_c)
        return accs

    for c0 in range(LEAD_P):
        issue_p(c0, c0)
    for c0 in range(LEAD_Q):
        drain_p()
        issue_q(c0, c0)

    def outer(i, accs):
        for k in range(PB):
            accs = body(i * PB + k, k, accs)
        return accs

    zero = jnp.zeros((L,), jnp.float32)
    accs = tuple(zero for _ in range(2 * G))
    accs = lax.fori_loop(0, CHUNKS_A // PB, outer, accs)
    for t in range((CHUNKS_A // PB) * PB, CHUNKS_A):
        accs = body(t, t % PB, accs)
    for _ in range(LEAD_P):
        drain_w()

    for g in range(G):
        stat_v[0, pl.ds(g * L, L)] = accs[g]
        stat_v[1, pl.ds(g * L, L)] = accs[G + g]
    pltpu.sync_copy(stat_v, out_hbm.at[wid])


@functools.partial(
    pl.kernel,
    mesh=_mesh,
    out_type=jax.ShapeDtypeStruct((NC, NPAD, C), jnp.float32),
    scratch_types=[
        pltpu.VMEM((NIDX, KB), jnp.int32),
        pltpu.VMEM((NBUF, KB, C), jnp.float32),
        pltpu.VMEM((NAB, KB, C), jnp.float32),
        pltpu.VMEM((2, C), jnp.float32),
        pltpu.VMEM_SHARED((NPAD, C), jnp.float32),
        pltpu.SemaphoreType.DMA,
        pltpu.SemaphoreType.DMA,
        pltpu.SemaphoreType.DMA,
    ],
)
def _edge_kernel(ri_hbm, h_hbm, ss_hbm, z_hbm, out_hbm,
                 idx, buf_h, act, ss_v, accum,
                 sem_i, sem_g, sem_s):
    cid = lax.axis_index("c")
    sid = lax.axis_index("s")
    wid = sid * NC + cid

    pltpu.sync_copy(ss_hbm, ss_v)
    svals = [ss_v[0, pl.ds(g * L, L)] for g in range(G)]
    tvals = [ss_v[1, pl.ds(g * L, L)] for g in range(G)]

    pltpu.sync_copy(
        z_hbm.at[pl.ds(sid * ROWS_PER_TILE, ROWS_PER_TILE)],
        accum.at[pl.ds(sid * ROWS_PER_TILE, ROWS_PER_TILE)],
    )
    plsc.subcore_barrier()

    def issue_idx(c, ib):
        pltpu.async_copy(ri_hbm.at[wid, c], idx.at[pl.ds(ib, 1)], sem_i)

    def drain_idx():
        pltpu.make_async_copy(ri_hbm.at[0, 0], idx.at[pl.ds(0, 1)],
                              sem_i).wait()

    def issue_read(c, b):
        base = pl.multiple_of(c * KB, KB)
        pltpu.async_copy(h_hbm.at[wid, pl.ds(base, KB)], buf_h.at[b], sem_g)

    def drain_read(b):
        pltpu.make_async_copy(
            h_hbm.at[0, pl.ds(0, KB)], buf_h.at[b], sem_g
        ).wait()

    def drain_scatter(sb):
        pltpu.make_async_copy(
            act.at[sb], accum.at[idx.at[0]], sem_s
        ).wait()

    def make_edge(b, sb):
        def edge(j, _):
            for g in range(G):
                sl = pl.ds(g * L, L)
                z = buf_h[b, j, sl] * svals[g] + tvals[g]
                act[sb, j, sl] = z / (-1.0 - jnp.exp(z))
            return 0

        return edge

    for c0 in range(NIDX):
        issue_idx(c0, c0)
    for c0 in range(NBUF):
        issue_read(c0, c0)

    def body(c, u):
        b = u % NBUF
        ib = u % NIDX
        sb = u % NAB

        drain_read(b)

        @pl.when(c >= NAB)
        def _():
            drain_scatter(sb)

            @pl.when(c + NIDX - NAB < CHUNKS_B)
            def _():
                issue_idx(c + NIDX - NAB, (c + NIDX - NAB) % NIDX)

        lax.fori_loop(0, KB, make_edge(b, sb), 0)

        @pl.when(c + NBUF < CHUNKS_B)
        def _():
            issue_read(c + NBUF, b)

        drain_idx()
        pltpu.async_copy(act.at[sb], accum.at[idx.at[ib]], sem_s, add=True)

    def outer(i, _):
        for u in range(NIDX):
            body(i * NIDX + u, u)
        return 0

    lax.fori_loop(0, CHUNKS_B // NIDX, outer, 0)
    for t in range((CHUNKS_B // NIDX) * NIDX, CHUNKS_B):
        body(t, t % NIDX)
    for sb in range(NAB):
        drain_scatter(sb)

    plsc.subcore_barrier()
    pltpu.sync_copy(
        accum.at[pl.ds(sid * ROWS_PER_TILE, ROWS_PER_TILE)],
        out_hbm.at[cid, pl.ds(sid * ROWS_PER_TILE, ROWS_PER_TILE)],
    )


def _fold_body(part_ref, out_ref):
    out_ref[...] = part_ref[0] + part_ref[1]


def _fold(partials):
    blk = 1000
    return pl.pallas_call(
        _fold_body,
        grid=(N // blk,),
        in_specs=[pl.BlockSpec((NC, blk, C), lambda i: (0, i, 0))],
        out_specs=pl.BlockSpec((blk, C), lambda i: (i, 0)),
        out_shape=jax.ShapeDtypeStruct((N, C), jnp.float32),
    )(partials)


def kernel(x_bk_c, edge_index_batched, W, gamma, beta):
    row3a = edge_index_batched[0].reshape(NW, CHUNKS_A, KA)
    col3a = edge_index_batched[1].reshape(NW, CHUNKS_A, KA)
    ri4 = edge_index_batched[0].reshape(NW, CHUNKS_B, 1, KB)
    p_tab, q_tab = _make_pq(x_bk_c, W)

    stats, h = _stats_kernel(row3a, col3a, p_tab, q_tab)
    ss = _ssmake(stats, gamma.reshape(1, C), beta.reshape(1, C))

    zeros = jnp.zeros((NPAD, C), jnp.float32)
    partials = _edge_kernel(ri4, h, ss, zeros)
    return _fold(partials)
